# baseline (speedup 1.0000x reference)
import functools

import jax
import jax.numpy as jnp
from jax import lax
from jax.experimental import pallas as pl
from jax.experimental.pallas import tpu as pltpu
from jax.experimental.pallas import tpu_sc as plsc

_N = 10000
_E = 320000
_F = 128
_NPAD = 10240
_R = _NPAD // 128
_NC = 2
_NS = 16
_NW = _NC * _NS
_C = 112
_NBUF = 3
_NCH = 93
_EPT = _NCH * _C
_EPAD = _EPT * _NW
_STRIPE = _NPAD // _NS

_HI = lax.Precision.HIGHEST


def _segsum(h, srcp, dstp):
  mesh = plsc.VectorSubcoreMesh(core_axis_name="c", subcore_axis_name="s")

  @functools.partial(
      pl.kernel,
      out_type=jax.ShapeDtypeStruct((_NC, _NPAD, _F), jnp.float32),
      mesh=mesh,
      scratch_types=[
          [pltpu.VMEM((_C,), jnp.int32)] * _NBUF,
          [pltpu.VMEM((_C,), jnp.int32)] * _NBUF,
          pltpu.VMEM((_NBUF, _C, _F), jnp.float32),
          pltpu.VMEM_SHARED((_NPAD, _F), jnp.float32),
          [pltpu.SemaphoreType.DMA] * _NBUF,
          [pltpu.SemaphoreType.DMA] * _NBUF,
      ],
  )
  def seg(h_hbm, src_hbm, dst_hbm, out_hbm, sidx, didx, rows, agg_sh,
          gsems, ssems):
    cid = lax.axis_index("c")
    sid = lax.axis_index("s")
    wid = cid * _NS + sid

    zero = jnp.zeros((16,), jnp.float32)

    def zrow(i, _):
      r = i // (_F // 16)
      c = (i % (_F // 16)) * 16
      rows[0, r, pl.ds(c, 16)] = zero
      return 0

    lax.fori_loop(0, _C * (_F // 16), zrow, 0)

    for i in range(_STRIPE // _C):
      pltpu.sync_copy(rows.at[0],
                      agg_sh.at[pl.ds(sid * _STRIPE + i * _C, _C)])
    _REM = _STRIPE % _C
    if _REM:
      pltpu.sync_copy(
          rows.at[0].at[pl.ds(0, _REM)],
          agg_sh.at[pl.ds(sid * _STRIPE + (_STRIPE // _C) * _C, _REM)])
    plsc.subcore_barrier()

    base = wid * _EPT

    def fetch(j, b):
      off = base + j * _C
      pltpu.sync_copy(src_hbm.at[pl.ds(off, _C)], sidx[b])
      pltpu.sync_copy(dst_hbm.at[pl.ds(off, _C)], didx[b])
      pltpu.async_copy(h_hbm.at[sidx[b]], rows.at[b], gsems[b])

    for b in range(_NBUF):
      fetch(b, b)

    def group(jj, _):
      for b in range(_NBUF):
        j = jj * _NBUF + b
        pltpu.make_async_copy(h_hbm.at[sidx[b]], rows.at[b],
                              gsems[b]).wait()
        pltpu.async_copy(rows.at[b], agg_sh.at[didx[b]], ssems[b],
                         add=True)
        pltpu.make_async_copy(rows.at[b], agg_sh.at[didx[b]],
                              ssems[b]).wait()
        fetch(j + _NBUF, b)
      return 0

    lax.fori_loop(0, _NCH // _NBUF - 1, group, 0)

    for b in range(_NBUF):
      pltpu.make_async_copy(h_hbm.at[sidx[b]], rows.at[b], gsems[b]).wait()
      pltpu.async_copy(rows.at[b], agg_sh.at[didx[b]], ssems[b], add=True)
    for b in range(_NBUF):
      pltpu.make_async_copy(rows.at[b], agg_sh.at[didx[b]],
                            ssems[b]).wait()

    plsc.subcore_barrier()
    pltpu.sync_copy(agg_sh.at[pl.ds(sid * _STRIPE, _STRIPE)],
                    out_hbm.at[cid].at[pl.ds(sid * _STRIPE, _STRIPE)])

  return seg(h, srcp, dstp)


def _convpool(agg2, h, wrel, brel, wroot, p, kept, k):
  def body(agg_ref, h_ref, wrel_ref, brel_ref, wroot_ref, p_ref, kept_ref,
           hnew_ref, keptn_ref, ro_ref):
    agg = agg_ref[0] + agg_ref[1]
    hh = h_ref[...]
    hp = (jnp.dot(agg, wrel_ref[...], precision=_HI)
          + jnp.dot(hh, wroot_ref[...], precision=_HI)
          + brel_ref[...])
    hp = jnp.maximum(hp, 0.0)

    pv = p_ref[...]
    pn = lax.rsqrt(jnp.sum(pv * pv))
    hp3 = hp.reshape(_R, 128, _F)
    score = jnp.sum(hp3 * pv[0][None, None, :], axis=2) * pn

    imin = jnp.int32(-2147483648)
    keptold = kept_ref[...]
    sc_sel = jnp.where(keptold > 0, score, -jnp.inf)
    bits = lax.bitcast_convert_type(sc_sel, jnp.int32)
    skey = jnp.where(bits >= 0, bits, -(bits & jnp.int32(0x7FFFFFFF)))

    def rbit(i, acc):
      trial = acc | lax.shift_left(jnp.int32(1), 31 - i)
      cnt = jnp.sum((skey >= (trial ^ imin)).astype(jnp.int32))
      return jnp.where(cnt >= k, trial, acc)

    tau = lax.fori_loop(0, 32, rbit, jnp.int32(0)) ^ imin

    gt = skey > tau
    cnt_gt = jnp.sum(gt.astype(jnp.int32))
    need = k - cnt_gt
    eq = skey == tau
    idx = (lax.broadcasted_iota(jnp.int32, (_R, 128), 0) * 128
           + lax.broadcasted_iota(jnp.int32, (_R, 128), 1))

    def dbit(i, acc):
      trial = acc | lax.shift_left(jnp.int32(1), 13 - i)
      d = jnp.sum((eq & (idx <= trial)).astype(jnp.int32))
      return jnp.where(d < need, trial, acc)

    tp = lax.fori_loop(0, 14, dbit, jnp.int32(0))
    d0 = jnp.sum((eq & (idx <= 0)).astype(jnp.int32))
    tf = jnp.where(d0 >= need, jnp.int32(0), tp + 1)
    keptn = gt | (eq & (idx <= tf))
    keptf = keptn.astype(jnp.float32)

    mult = jnp.tanh(score) * keptf
    hnew3 = hp3 * mult[:, :, None]
    hnew_ref[...] = hnew3.reshape(_NPAD, _F)
    keptn_ref[...] = keptf

    mx3 = hnew3 + (keptf[:, :, None] - 1.0) * jnp.float32(3.0e38)
    mx = jnp.max(jnp.max(mx3, axis=0), axis=0)
    sm = jnp.sum(jnp.sum(hnew3, axis=0), axis=0) * jnp.float32(1.0 / k)
    ro_ref[:, pl.ds(0, _F)] = mx[None, :]
    ro_ref[:, pl.ds(_F, _F)] = sm[None, :]

  return pl.pallas_call(
      body,
      out_shape=[
          jax.ShapeDtypeStruct((_NPAD, _F), jnp.float32),
          jax.ShapeDtypeStruct((_R, 128), jnp.float32),
          jax.ShapeDtypeStruct((1, 2 * _F), jnp.float32),
      ],
  )(agg2, h, wrel, brel, wroot, p, kept)


def _mlp(x1, x2, x3, w1, b1, w2, b2, w3, b3):
  def body(x1r, x2r, x3r, w1r, b1r, w2r, b2r, w3r, b3r, out_ref):
    z = x1r[...] + x2r[...] + x3r[...]
    z = jnp.maximum(jnp.dot(z, w1r[...], precision=_HI) + b1r[...], 0.0)
    z = jnp.maximum(jnp.dot(z, w2r[...], precision=_HI) + b2r[...], 0.0)
    z = jnp.dot(z, w3r[...], precision=_HI) + b3r[...]
    out_ref[...] = 1.0 / (1.0 + jnp.exp(-z))

  return pl.pallas_call(
      body,
      out_shape=jax.ShapeDtypeStruct((1, 1), jnp.float32),
  )(x1, x2, x3, w1, b1, w2, b2, w3, b3)


def kernel(x, edge_index, batch, W_rel1, b_rel1, W_root1, p1, W_rel2, b_rel2,
           W_root2, p2, W_rel3, b_rel3, W_root3, p3, W_lin1, b_lin1, W_lin2,
           b_lin2, W_lin3, b_lin3):
  src, dst = edge_index[0], edge_index[1]
  pad = _N + jnp.arange(_EPAD - _E, dtype=jnp.int32) % (_NPAD - _N)
  srcp = jnp.concatenate([src, pad])
  dstp = jnp.concatenate([dst, pad])
  xp = jnp.concatenate(
      [x, jnp.zeros((_NPAD - _N, _F), jnp.float32)], axis=0)
  kept0 = (jnp.arange(_NPAD, dtype=jnp.int32) < _N).astype(
      jnp.float32).reshape(_R, 128)

  k1, k2, k3 = 8000, 6400, 5120

  agg = _segsum(xp, srcp, dstp)
  h, kept1, x1 = _convpool(agg, xp, W_rel1, b_rel1[None, :], W_root1,
                           p1[None, :], kept0, k1)
  agg = _segsum(h, srcp, dstp)
  h, kept2, x2 = _convpool(agg, h, W_rel2, b_rel2[None, :], W_root2,
                           p2[None, :], kept1, k2)
  agg = _segsum(h, srcp, dstp)
  h, kept3, x3 = _convpool(agg, h, W_rel3, b_rel3[None, :], W_root3,
                           p3[None, :], kept2, k3)

  return _mlp(x1, x2, x3, W_lin1, b_lin1[None, :], W_lin2, b_lin2[None, :],
              W_lin3, b_lin3[None, :])


# speedup vs baseline: 1.0488x; 1.0488x over previous
import functools

import jax
import jax.numpy as jnp
from jax import lax
from jax.experimental import pallas as pl
from jax.experimental.pallas import tpu as pltpu
from jax.experimental.pallas import tpu_sc as plsc

_N = 10000
_E = 320000
_F = 128
_NPAD = 10240
_R = _NPAD // 128
_NC = 2
_NS = 16
_NW = _NC * _NS
_C = 128
_NBUF = 2
_NCH = 80
_EPT = _NCH * _C
_EPAD = _EPT * _NW
_STRIPE = _NPAD // _NS

_HI = lax.Precision.HIGHEST


def _segsum(h, srcp, dstp):
  mesh = plsc.VectorSubcoreMesh(core_axis_name="c", subcore_axis_name="s")

  @functools.partial(
      pl.kernel,
      out_type=jax.ShapeDtypeStruct((_NC, _NPAD, _F), jnp.float32),
      mesh=mesh,
      scratch_types=[
          [pltpu.VMEM((_C,), jnp.int32)] * _NBUF,
          [pltpu.VMEM((_C,), jnp.int32)] * _NBUF,
          pltpu.VMEM((_NBUF, _C, _F), jnp.float32),
          pltpu.VMEM_SHARED((_NPAD, _F), jnp.float32),
          [pltpu.SemaphoreType.DMA] * _NBUF,
          [pltpu.SemaphoreType.DMA] * _NBUF,
      ],
  )
  def seg(h_hbm, src_hbm, dst_hbm, out_hbm, sidx, didx, rows, agg_sh,
          gsems, ssems):
    cid = lax.axis_index("c")
    sid = lax.axis_index("s")
    wid = cid * _NS + sid

    zero = jnp.zeros((16,), jnp.float32)

    def zrow(i, _):
      r = i // (_F // 16)
      c = (i % (_F // 16)) * 16
      rows[0, r, pl.ds(c, 16)] = zero
      return 0

    lax.fori_loop(0, _C * (_F // 16), zrow, 0)

    def zstripe(j, _):
      pltpu.sync_copy(rows.at[0],
                      agg_sh.at[pl.ds(sid * _STRIPE + j * _C, _C)])
      return 0

    lax.fori_loop(0, _STRIPE // _C, zstripe, 0)
    plsc.subcore_barrier()

    base = wid * _EPT

    def fetch(j, b):
      off = base + j * _C
      pltpu.sync_copy(src_hbm.at[pl.ds(off, _C)], sidx[b])
      pltpu.sync_copy(dst_hbm.at[pl.ds(off, _C)], didx[b])
      pltpu.async_copy(h_hbm.at[sidx[b]], rows.at[b], gsems[b])

    for b in range(_NBUF):
      fetch(b, b)

    def group(jj, _):
      for b in range(_NBUF):
        j = jj * _NBUF + b
        pltpu.make_async_copy(h_hbm.at[sidx[b]], rows.at[b],
                              gsems[b]).wait()
        pltpu.async_copy(rows.at[b], agg_sh.at[didx[b]], ssems[b],
                         add=True)
        pltpu.make_async_copy(rows.at[b], agg_sh.at[didx[b]],
                              ssems[b]).wait()
        fetch(j + _NBUF, b)
      return 0

    lax.fori_loop(0, _NCH // _NBUF - 1, group, 0)

    for b in range(_NBUF):
      pltpu.make_async_copy(h_hbm.at[sidx[b]], rows.at[b], gsems[b]).wait()
      pltpu.async_copy(rows.at[b], agg_sh.at[didx[b]], ssems[b], add=True)
    for b in range(_NBUF):
      pltpu.make_async_copy(rows.at[b], agg_sh.at[didx[b]],
                            ssems[b]).wait()

    plsc.subcore_barrier()
    pltpu.sync_copy(agg_sh.at[pl.ds(sid * _STRIPE, _STRIPE)],
                    out_hbm.at[cid].at[pl.ds(sid * _STRIPE, _STRIPE)])

  return seg(h, srcp, dstp)


def _convpool(agg2, h, wrel, brel, wroot, p, kept, k):
  def body(agg_ref, h_ref, wrel_ref, brel_ref, wroot_ref, p_ref, kept_ref,
           hnew_ref, keptn_ref, ro_ref):
    agg = agg_ref[0] + agg_ref[1]
    hh = h_ref[...]
    hp = (jnp.dot(agg, wrel_ref[...], precision=_HI)
          + jnp.dot(hh, wroot_ref[...], precision=_HI)
          + brel_ref[...])
    hp = jnp.maximum(hp, 0.0)

    pv = p_ref[...]
    pn = lax.rsqrt(jnp.sum(pv * pv))
    hp3 = hp.reshape(_R, 128, _F)
    score = jnp.sum(hp3 * pv[0][None, None, :], axis=2) * pn

    imin = jnp.int32(-2147483648)
    keptold = kept_ref[...]
    sc_sel = jnp.where(keptold > 0, score, -jnp.inf)
    bits = lax.bitcast_convert_type(sc_sel, jnp.int32)
    skey = jnp.where(bits >= 0, bits, -(bits & jnp.int32(0x7FFFFFFF)))

    def rbit(i, acc):
      trial = acc | lax.shift_left(jnp.int32(1), 31 - i)
      cnt = jnp.sum((skey >= (trial ^ imin)).astype(jnp.int32))
      return jnp.where(cnt >= k, trial, acc)

    tau = lax.fori_loop(0, 32, rbit, jnp.int32(0)) ^ imin

    gt = skey > tau
    cnt_gt = jnp.sum(gt.astype(jnp.int32))
    need = k - cnt_gt
    eq = skey == tau
    idx = (lax.broadcasted_iota(jnp.int32, (_R, 128), 0) * 128
           + lax.broadcasted_iota(jnp.int32, (_R, 128), 1))

    def dbit(i, acc):
      trial = acc | lax.shift_left(jnp.int32(1), 13 - i)
      d = jnp.sum((eq & (idx <= trial)).astype(jnp.int32))
      return jnp.where(d < need, trial, acc)

    tp = lax.fori_loop(0, 14, dbit, jnp.int32(0))
    d0 = jnp.sum((eq & (idx <= 0)).astype(jnp.int32))
    tf = jnp.where(d0 >= need, jnp.int32(0), tp + 1)
    keptn = gt | (eq & (idx <= tf))
    keptf = keptn.astype(jnp.float32)

    mult = jnp.tanh(score) * keptf
    hnew3 = hp3 * mult[:, :, None]
    hnew_ref[...] = hnew3.reshape(_NPAD, _F)
    keptn_ref[...] = keptf

    mx3 = hnew3 + (keptf[:, :, None] - 1.0) * jnp.float32(3.0e38)
    mx = jnp.max(jnp.max(mx3, axis=0), axis=0)
    sm = jnp.sum(jnp.sum(hnew3, axis=0), axis=0) * jnp.float32(1.0 / k)
    ro_ref[:, pl.ds(0, _F)] = mx[None, :]
    ro_ref[:, pl.ds(_F, _F)] = sm[None, :]

  return pl.pallas_call(
      body,
      out_shape=[
          jax.ShapeDtypeStruct((_NPAD, _F), jnp.float32),
          jax.ShapeDtypeStruct((_R, 128), jnp.float32),
          jax.ShapeDtypeStruct((1, 2 * _F), jnp.float32),
      ],
  )(agg2, h, wrel, brel, wroot, p, kept)


def _mlp(x1, x2, x3, w1, b1, w2, b2, w3, b3):
  def body(x1r, x2r, x3r, w1r, b1r, w2r, b2r, w3r, b3r, out_ref):
    z = x1r[...] + x2r[...] + x3r[...]
    z = jnp.maximum(jnp.dot(z, w1r[...], precision=_HI) + b1r[...], 0.0)
    z = jnp.maximum(jnp.dot(z, w2r[...], precision=_HI) + b2r[...], 0.0)
    z = jnp.dot(z, w3r[...], precision=_HI) + b3r[...]
    out_ref[...] = 1.0 / (1.0 + jnp.exp(-z))

  return pl.pallas_call(
      body,
      out_shape=jax.ShapeDtypeStruct((1, 1), jnp.float32),
  )(x1, x2, x3, w1, b1, w2, b2, w3, b3)


def kernel(x, edge_index, batch, W_rel1, b_rel1, W_root1, p1, W_rel2, b_rel2,
           W_root2, p2, W_rel3, b_rel3, W_root3, p3, W_lin1, b_lin1, W_lin2,
           b_lin2, W_lin3, b_lin3):
  src, dst = edge_index[0], edge_index[1]
  pad = _N + jnp.arange(_EPAD - _E, dtype=jnp.int32) % (_NPAD - _N)
  srcp = jnp.concatenate([src, pad])
  dstp = jnp.concatenate([dst, pad])
  xp = jnp.concatenate(
      [x, jnp.zeros((_NPAD - _N, _F), jnp.float32)], axis=0)
  kept0 = (jnp.arange(_NPAD, dtype=jnp.int32) < _N).astype(
      jnp.float32).reshape(_R, 128)

  k1, k2, k3 = 8000, 6400, 5120

  agg = _segsum(xp, srcp, dstp)
  h, kept1, x1 = _convpool(agg, xp, W_rel1, b_rel1[None, :], W_root1,
                           p1[None, :], kept0, k1)
  agg = _segsum(h, srcp, dstp)
  h, kept2, x2 = _convpool(agg, h, W_rel2, b_rel2[None, :], W_root2,
                           p2[None, :], kept1, k2)
  agg = _segsum(h, srcp, dstp)
  h, kept3, x3 = _convpool(agg, h, W_rel3, b_rel3[None, :], W_root3,
                           p3[None, :], kept2, k3)

  return _mlp(x1, x2, x3, W_lin1, b_lin1[None, :], W_lin2, b_lin2[None, :],
              W_lin3, b_lin3[None, :])
